# baseline (speedup 1.0000x reference)
import jax
import jax.numpy as jnp
from jax import lax
from jax.experimental import pallas as pl
from jax.experimental.pallas import tpu as pltpu

B, P, D, H = 512, 256, 128, 192
TB = 16


def _fused_kernel(x_ref, w1t_ref, b1_ref, w2t_ref, b2_ref, y_ref, m_ref, p_ref):
    R = TB * P
    xall = x_ref[...].reshape(R, D)
    xt = xall.T

    xtb = xt.astype(jnp.bfloat16)
    ht = lax.dot_general(w1t_ref[...], xtb, (((1,), (0,)), ((), ())),
                         preferred_element_type=jnp.float32)
    hb = (ht + b1_ref[...]).astype(jnp.bfloat16)
    one = jnp.bfloat16(1.0)
    half = jnp.bfloat16(0.5)
    isq2 = jnp.bfloat16(0.7071067811865476)
    g = half * hb * (one + lax.erf(hb * isq2))
    pt = lax.dot_general(w2t_ref[...], g, (((1,), (0,)), ((), ())),
                         preferred_element_type=jnp.float32)
    pt = pt + b2_ref[...]

    nz = jnp.sum((xt[:10, :] != 0.0).astype(jnp.float32), axis=0,
                 keepdims=True)
    yt = jnp.where(nz > 0.0, pt, 0.0)

    p7 = xt[:7, :]
    idx = lax.broadcasted_iota(jnp.int32, (7, R), 0)
    p7 = jnp.where(idx < 4, p7, 0.0)
    p7 = jnp.where(idx == 5, 1.0, p7)

    for b in range(TB):
        lo, hi = b * P, (b + 1) * P
        y_ref[b] = yt[:, lo:hi]
        m_ref[b:b + 1, :] = nz[:, lo:hi] == 0.0
        p_ref[:, b, :] = p7[:, lo:hi]


def kernel(x, W1, b1, W2, b2):
    w1t = W1.T.astype(jnp.bfloat16)
    w2t = W2.T.astype(jnp.bfloat16)
    b1c = b1.reshape(H, 1)
    b2c = b2.reshape(H, 1)
    grid = (B // TB,)
    yt, mask, post = pl.pallas_call(
        _fused_kernel,
        grid=grid,
        in_specs=[
            pl.BlockSpec((TB, P, D), lambda i: (i, 0, 0)),
            pl.BlockSpec((H, D), lambda i: (0, 0)),
            pl.BlockSpec((H, 1), lambda i: (0, 0)),
            pl.BlockSpec((H, H), lambda i: (0, 0)),
            pl.BlockSpec((H, 1), lambda i: (0, 0)),
        ],
        out_specs=[
            pl.BlockSpec((TB, H, P), lambda i: (i, 0, 0)),
            pl.BlockSpec((TB, P), lambda i: (i, 0)),
            pl.BlockSpec((7, TB, P), lambda i: (0, i, 0)),
        ],
        out_shape=[
            jax.ShapeDtypeStruct((B, H, P), jnp.float32),
            jax.ShapeDtypeStruct((B, P), jnp.bool_),
            jax.ShapeDtypeStruct((7, B, P), jnp.float32),
        ],
        compiler_params=pltpu.CompilerParams(
            dimension_semantics=("parallel",),
        ),
    )(x, w1t, b1c, w2t, b2c)
    y = jnp.transpose(yt, (0, 2, 1))
    pos = jnp.transpose(post, (1, 2, 0))
    return (y, mask, pos)


# speedup vs baseline: 1.1665x; 1.1665x over previous
import jax
import jax.numpy as jnp
from jax import lax
from jax.experimental import pallas as pl
from jax.experimental.pallas import tpu as pltpu

B, P, D, H = 512, 256, 128, 192
TB = 64


def _fused_kernel(x_ref, w1t_ref, b1_ref, w2t_ref, b2_ref, y_ref, m_ref, p_ref):
    R = TB * P
    xall = x_ref[...].reshape(R, D)
    xt = xall.T

    xtb = xt.astype(jnp.bfloat16)
    ht = lax.dot_general(w1t_ref[...], xtb, (((1,), (0,)), ((), ())),
                         preferred_element_type=jnp.float32)
    hb = (ht + b1_ref[...]).astype(jnp.bfloat16)
    one = jnp.bfloat16(1.0)
    half = jnp.bfloat16(0.5)
    isq2 = jnp.bfloat16(0.7071067811865476)
    g = half * hb * (one + lax.erf(hb * isq2))
    pt = lax.dot_general(w2t_ref[...], g, (((1,), (0,)), ((), ())),
                         preferred_element_type=jnp.float32)
    pt = pt + b2_ref[...]

    nz = jnp.sum((xt[:10, :] != 0.0).astype(jnp.float32), axis=0,
                 keepdims=True)
    yt = jnp.where(nz > 0.0, pt, 0.0)

    p7 = xt[:7, :]
    idx = lax.broadcasted_iota(jnp.int32, (7, R), 0)
    p7 = jnp.where(idx < 4, p7, 0.0)
    p7 = jnp.where(idx == 5, 1.0, p7)

    for b in range(TB):
        lo, hi = b * P, (b + 1) * P
        y_ref[b] = yt[:, lo:hi]
        m_ref[b:b + 1, :] = nz[:, lo:hi] == 0.0
        p_ref[:, b, :] = p7[:, lo:hi]


def kernel(x, W1, b1, W2, b2):
    w1t = W1.T.astype(jnp.bfloat16)
    w2t = W2.T.astype(jnp.bfloat16)
    b1c = b1.reshape(H, 1)
    b2c = b2.reshape(H, 1)
    grid = (B // TB,)
    yt, mask, post = pl.pallas_call(
        _fused_kernel,
        grid=grid,
        in_specs=[
            pl.BlockSpec((TB, P, D), lambda i: (i, 0, 0)),
            pl.BlockSpec((H, D), lambda i: (0, 0)),
            pl.BlockSpec((H, 1), lambda i: (0, 0)),
            pl.BlockSpec((H, H), lambda i: (0, 0)),
            pl.BlockSpec((H, 1), lambda i: (0, 0)),
        ],
        out_specs=[
            pl.BlockSpec((TB, H, P), lambda i: (i, 0, 0)),
            pl.BlockSpec((TB, P), lambda i: (i, 0)),
            pl.BlockSpec((7, TB, P), lambda i: (0, i, 0)),
        ],
        out_shape=[
            jax.ShapeDtypeStruct((B, H, P), jnp.float32),
            jax.ShapeDtypeStruct((B, P), jnp.bool_),
            jax.ShapeDtypeStruct((7, B, P), jnp.float32),
        ],
        compiler_params=pltpu.CompilerParams(
            dimension_semantics=("parallel",),
        ),
    )(x, w1t, b1c, w2t, b2c)
    y = jnp.transpose(yt, (0, 2, 1))
    pos = jnp.transpose(post, (1, 2, 0))
    return (y, mask, pos)


# speedup vs baseline: 1.1715x; 1.0043x over previous
import jax
import jax.numpy as jnp
from jax import lax
from jax.experimental import pallas as pl
from jax.experimental.pallas import tpu as pltpu

B, P, D, H = 512, 256, 128, 192
TB = 64


def _fused_kernel(x_ref, w1_ref, b1_ref, w2_ref, b2_ref, y_ref, m_ref, p_ref):
    R = TB * P
    xall = x_ref[...].reshape(R, D)
    xt = xall.T

    xtb = xt.astype(jnp.bfloat16)
    w1t = w1_ref[...].astype(jnp.bfloat16).T
    w2t = w2_ref[...].astype(jnp.bfloat16).T
    ht = lax.dot_general(w1t, xtb, (((1,), (0,)), ((), ())),
                         preferred_element_type=jnp.float32)
    hb = (ht + b1_ref[...]).astype(jnp.bfloat16)
    one = jnp.bfloat16(1.0)
    half = jnp.bfloat16(0.5)
    isq2 = jnp.bfloat16(0.7071067811865476)
    g = half * hb * (one + lax.erf(hb * isq2))
    pt = lax.dot_general(w2t, g, (((1,), (0,)), ((), ())),
                         preferred_element_type=jnp.float32)
    pt = pt + b2_ref[...]

    nz = jnp.sum((xt[:10, :] != 0.0).astype(jnp.float32), axis=0,
                 keepdims=True)
    yt = jnp.where(nz > 0.0, pt, 0.0)

    p7 = xt[:7, :]
    idx = lax.broadcasted_iota(jnp.int32, (7, R), 0)
    p7 = jnp.where(idx < 4, p7, 0.0)
    p7 = jnp.where(idx == 5, 1.0, p7)

    for b in range(TB):
        lo, hi = b * P, (b + 1) * P
        y_ref[b] = yt[:, lo:hi]
        m_ref[b:b + 1, :] = nz[:, lo:hi] == 0.0
        p_ref[:, b, :] = p7[:, lo:hi]


def kernel(x, W1, b1, W2, b2):
    b1c = b1.reshape(H, 1)
    b2c = b2.reshape(H, 1)
    grid = (B // TB,)
    yt, mask, post = pl.pallas_call(
        _fused_kernel,
        grid=grid,
        in_specs=[
            pl.BlockSpec((TB, P, D), lambda i: (i, 0, 0)),
            pl.BlockSpec((D, H), lambda i: (0, 0)),
            pl.BlockSpec((H, 1), lambda i: (0, 0)),
            pl.BlockSpec((H, H), lambda i: (0, 0)),
            pl.BlockSpec((H, 1), lambda i: (0, 0)),
        ],
        out_specs=[
            pl.BlockSpec((TB, H, P), lambda i: (i, 0, 0)),
            pl.BlockSpec((TB, P), lambda i: (i, 0)),
            pl.BlockSpec((7, TB, P), lambda i: (0, i, 0)),
        ],
        out_shape=[
            jax.ShapeDtypeStruct((B, H, P), jnp.float32),
            jax.ShapeDtypeStruct((B, P), jnp.bool_),
            jax.ShapeDtypeStruct((7, B, P), jnp.float32),
        ],
        compiler_params=pltpu.CompilerParams(
            dimension_semantics=("parallel",),
        ),
    )(x, W1, b1c, W2, b2c)
    y = jnp.transpose(yt, (0, 2, 1))
    pos = jnp.transpose(post, (1, 2, 0))
    return (y, mask, pos)


# speedup vs baseline: 1.2269x; 1.0472x over previous
import jax
import jax.numpy as jnp
from jax import lax
from jax.experimental import pallas as pl
from jax.experimental.pallas import tpu as pltpu

B, P, D, H = 512, 256, 128, 192
TB = 64


def _fused_kernel(x_ref, w1t_ref, b1_ref, w2t_ref, b2_ref, y_ref, m_ref, p_ref):
    R = TB * P
    xall = x_ref[...].reshape(R, D)
    xt = xall.T

    xtb = xt.astype(jnp.bfloat16)
    w1t = w1t_ref[...].astype(jnp.bfloat16)
    w2t = w2t_ref[...].astype(jnp.bfloat16)
    b1c = b1_ref[...].T
    b2c = b2_ref[...].T
    ht = lax.dot_general(w1t, xtb, (((1,), (0,)), ((), ())),
                         preferred_element_type=jnp.float32)
    hb = (ht + b1c).astype(jnp.bfloat16)
    one = jnp.bfloat16(1.0)
    half = jnp.bfloat16(0.5)
    isq2 = jnp.bfloat16(0.7071067811865476)
    g = half * hb * (one + lax.erf(hb * isq2))
    pt = lax.dot_general(w2t, g, (((1,), (0,)), ((), ())),
                         preferred_element_type=jnp.float32)
    pt = pt + b2c

    nz = jnp.sum((xt[:10, :] != 0.0).astype(jnp.float32), axis=0,
                 keepdims=True)
    yt = jnp.where(nz > 0.0, pt, 0.0)

    p7 = xt[:7, :]
    idx = lax.broadcasted_iota(jnp.int32, (7, R), 0)
    p7 = jnp.where(idx < 4, p7, 0.0)
    p7 = jnp.where(idx == 5, 1.0, p7)

    for b in range(TB):
        lo, hi = b * P, (b + 1) * P
        y_ref[b] = yt[:, lo:hi]
        m_ref[b:b + 1, :] = nz[:, lo:hi] == 0.0
        p_ref[:, b, :] = p7[:, lo:hi]


def kernel(x, W1, b1, W2, b2):
    w1t = W1.T
    w2t = W2.T
    b1r = b1.reshape(1, H)
    b2r = b2.reshape(1, H)
    grid = (B // TB,)
    yt, mask, post = pl.pallas_call(
        _fused_kernel,
        grid=grid,
        in_specs=[
            pl.BlockSpec((TB, P, D), lambda i: (i, 0, 0)),
            pl.BlockSpec((H, D), lambda i: (0, 0)),
            pl.BlockSpec((1, H), lambda i: (0, 0)),
            pl.BlockSpec((H, H), lambda i: (0, 0)),
            pl.BlockSpec((1, H), lambda i: (0, 0)),
        ],
        out_specs=[
            pl.BlockSpec((TB, H, P), lambda i: (i, 0, 0)),
            pl.BlockSpec((TB, P), lambda i: (i, 0)),
            pl.BlockSpec((7, TB, P), lambda i: (0, i, 0)),
        ],
        out_shape=[
            jax.ShapeDtypeStruct((B, H, P), jnp.float32),
            jax.ShapeDtypeStruct((B, P), jnp.bool_),
            jax.ShapeDtypeStruct((7, B, P), jnp.float32),
        ],
        compiler_params=pltpu.CompilerParams(
            dimension_semantics=("parallel",),
        ),
    )(x, w1t, b1r, w2t, b2r)
    y = jnp.transpose(yt, (0, 2, 1))
    pos = jnp.transpose(post, (1, 2, 0))
    return (y, mask, pos)


# speedup vs baseline: 1.2541x; 1.0222x over previous
import jax
import jax.numpy as jnp
from jax import lax
from jax.experimental import pallas as pl
from jax.experimental.pallas import tpu as pltpu

B, P, D, H = 512, 256, 128, 192
TB = 64


def _fused_kernel(x_ref, w1t_ref, b1_ref, w2_ref, b2_ref, y_ref, m_ref, p_ref):
    R = TB * P
    xall = x_ref[...].reshape(R, D)
    xt = xall.T

    xtb = xt.astype(jnp.bfloat16)
    w1t = w1t_ref[...].astype(jnp.bfloat16)
    w2b = w2_ref[...].astype(jnp.bfloat16)
    b1c = b1_ref[...].T
    b2c = b2_ref[...].T
    ht = lax.dot_general(w1t, xtb, (((1,), (0,)), ((), ())),
                         preferred_element_type=jnp.float32)
    hb = (ht + b1c).astype(jnp.bfloat16)
    one = jnp.bfloat16(1.0)
    half = jnp.bfloat16(0.5)
    isq2 = jnp.bfloat16(0.7071067811865476)
    g = half * hb * (one + lax.erf(hb * isq2))
    pt = lax.dot_general(w2b, g, (((0,), (0,)), ((), ())),
                         preferred_element_type=jnp.float32)
    pt = pt + b2c

    nz = jnp.sum((xt[:10, :] != 0.0).astype(jnp.float32), axis=0,
                 keepdims=True)
    yt = jnp.where(nz > 0.0, pt, 0.0)

    p7 = xt[:7, :]
    idx = lax.broadcasted_iota(jnp.int32, (7, R), 0)
    p7 = jnp.where(idx < 4, p7, 0.0)
    p7 = jnp.where(idx == 5, 1.0, p7)

    for b in range(TB):
        lo, hi = b * P, (b + 1) * P
        y_ref[b] = yt[:, lo:hi]
        m_ref[b:b + 1, :] = nz[:, lo:hi] == 0.0
        p_ref[:, b, :] = p7[:, lo:hi]


def kernel(x, W1, b1, W2, b2):
    w1t = W1.T
    b1r = b1.reshape(1, H)
    b2r = b2.reshape(1, H)
    grid = (B // TB,)
    yt, mask, post = pl.pallas_call(
        _fused_kernel,
        grid=grid,
        in_specs=[
            pl.BlockSpec((TB, P, D), lambda i: (i, 0, 0)),
            pl.BlockSpec((H, D), lambda i: (0, 0)),
            pl.BlockSpec((1, H), lambda i: (0, 0)),
            pl.BlockSpec((H, H), lambda i: (0, 0)),
            pl.BlockSpec((1, H), lambda i: (0, 0)),
        ],
        out_specs=[
            pl.BlockSpec((TB, H, P), lambda i: (i, 0, 0)),
            pl.BlockSpec((TB, P), lambda i: (i, 0)),
            pl.BlockSpec((7, TB, P), lambda i: (0, i, 0)),
        ],
        out_shape=[
            jax.ShapeDtypeStruct((B, H, P), jnp.float32),
            jax.ShapeDtypeStruct((B, P), jnp.bool_),
            jax.ShapeDtypeStruct((7, B, P), jnp.float32),
        ],
        compiler_params=pltpu.CompilerParams(
            dimension_semantics=("parallel",),
        ),
    )(x, w1t, b1r, W2, b2r)
    y = jnp.transpose(yt, (0, 2, 1))
    pos = jnp.transpose(post, (1, 2, 0))
    return (y, mask, pos)


# speedup vs baseline: 1.3628x; 1.0867x over previous
import jax
import jax.numpy as jnp
from jax import lax
from jax.experimental import pallas as pl
from jax.experimental.pallas import tpu as pltpu

B, P, D, H = 512, 256, 128, 192
TB = 64


def _fused_kernel(x_ref, w1t_ref, b1_ref, w2_ref, b2_ref, y_ref, m_ref, p_ref):
    R = TB * P
    xall = x_ref[...].reshape(R, D)
    xtb = xall.astype(jnp.bfloat16).T
    xt = xall[:, :16].T
    w1t = w1t_ref[...].astype(jnp.bfloat16)
    w2b = w2_ref[...].astype(jnp.bfloat16)
    b1c = b1_ref[...].T
    b2c = b2_ref[...].T
    ht = lax.dot_general(w1t, xtb, (((1,), (0,)), ((), ())),
                         preferred_element_type=jnp.float32)
    hb = (ht + b1c).astype(jnp.bfloat16)
    one = jnp.bfloat16(1.0)
    half = jnp.bfloat16(0.5)
    isq2 = jnp.bfloat16(0.7071067811865476)
    g = half * hb * (one + lax.erf(hb * isq2))
    pt = lax.dot_general(w2b, g, (((0,), (0,)), ((), ())),
                         preferred_element_type=jnp.float32)
    pt = pt + b2c

    nz = jnp.sum((xt[:10, :] != 0.0).astype(jnp.float32), axis=0,
                 keepdims=True)
    yt = jnp.where(nz > 0.0, pt, 0.0)

    p7 = xt[:7, :]
    idx = lax.broadcasted_iota(jnp.int32, (7, R), 0)
    p7 = jnp.where(idx < 4, p7, 0.0)
    p7 = jnp.where(idx == 5, 1.0, p7)

    for b in range(TB):
        lo, hi = b * P, (b + 1) * P
        y_ref[b] = yt[:, lo:hi]
        m_ref[b:b + 1, :] = nz[:, lo:hi] == 0.0
        p_ref[:, b, :] = p7[:, lo:hi]


def kernel(x, W1, b1, W2, b2):
    w1t = W1.T
    b1r = b1.reshape(1, H)
    b2r = b2.reshape(1, H)
    grid = (B // TB,)
    yt, mask, post = pl.pallas_call(
        _fused_kernel,
        grid=grid,
        in_specs=[
            pl.BlockSpec((TB, P, D), lambda i: (i, 0, 0)),
            pl.BlockSpec((H, D), lambda i: (0, 0)),
            pl.BlockSpec((1, H), lambda i: (0, 0)),
            pl.BlockSpec((H, H), lambda i: (0, 0)),
            pl.BlockSpec((1, H), lambda i: (0, 0)),
        ],
        out_specs=[
            pl.BlockSpec((TB, H, P), lambda i: (i, 0, 0)),
            pl.BlockSpec((TB, P), lambda i: (i, 0)),
            pl.BlockSpec((7, TB, P), lambda i: (0, i, 0)),
        ],
        out_shape=[
            jax.ShapeDtypeStruct((B, H, P), jnp.float32),
            jax.ShapeDtypeStruct((B, P), jnp.bool_),
            jax.ShapeDtypeStruct((7, B, P), jnp.float32),
        ],
        compiler_params=pltpu.CompilerParams(
            dimension_semantics=("parallel",),
        ),
    )(x, w1t, b1r, W2, b2r)
    y = jnp.transpose(yt, (0, 2, 1))
    pos = jnp.transpose(post, (1, 2, 0))
    return (y, mask, pos)
